# baseline (speedup 1.0000x reference)
import jax
import jax.numpy as jnp
from jax.experimental import pallas as pl
from jax.experimental.pallas import tpu as pltpu

BI = 200


def _gcn_kernel(feat_ref, adj_ref, w_ref, b_ref, out_ref, support_ref):
    i = pl.program_id(0)

    @pl.when(i == 0)
    def _():
        support_ref[...] = jnp.dot(
            feat_ref[...], w_ref[...], preferred_element_type=jnp.float32
        )

    out_ref[...] = (
        jnp.dot(adj_ref[...], support_ref[...], preferred_element_type=jnp.float32)
        + b_ref[...]
    )


def kernel(feat, adj, W, b):
    N, din = feat.shape
    dout = W.shape[1]
    b2 = b.reshape(1, dout)
    grid = (pl.cdiv(N, BI),)
    return pl.pallas_call(
        _gcn_kernel,
        grid=grid,
        in_specs=[
            pl.BlockSpec((N, din), lambda i: (0, 0)),
            pl.BlockSpec((BI, N), lambda i: (i, 0)),
            pl.BlockSpec((din, dout), lambda i: (0, 0)),
            pl.BlockSpec((1, dout), lambda i: (0, 0)),
        ],
        out_specs=pl.BlockSpec((BI, dout), lambda i: (i, 0)),
        out_shape=jax.ShapeDtypeStruct((N, dout), jnp.float32),
        scratch_shapes=[pltpu.VMEM((N, dout), jnp.float32)],
    )(feat, adj, W, b2)
